# baseline (speedup 1.0000x reference)
import functools

import jax
import jax.numpy as jnp
from jax import lax
from jax.experimental import pallas as pl
from jax.experimental.pallas import tpu as pltpu
from jax.experimental.pallas import tpu_sc as plsc

_NC, _NS = 2, 16
_NW = _NC * _NS
_BT = 128
_OB = 4
_L = 16
_PITCH = 129
_SH = 4

_BCAST_DN = lax.GatherDimensionNumbers(
    offset_dims=(), collapsed_slice_dims=(0,), start_index_map=(0,))


def _bcast_lane(vec, lane):
    idx = jnp.full((_L, 1), lane, jnp.int32)
    return lax.gather(vec, idx, _BCAST_DN, slice_sizes=(1,),
                      mode=lax.GatherScatterMode.PROMISE_IN_BOUNDS)


def _embed_gather(idx4, table_flat, depth):
    nst, nbt, sst, _ = idx4.shape
    seq = nst * sst
    ndt = depth // 8

    mesh = plsc.VectorSubcoreMesh(core_axis_name="c", subcore_axis_name="s")

    @functools.partial(
        pl.kernel,
        out_type=jax.ShapeDtypeStruct((seq, ndt, _NW, 8, _BT), jnp.float32),
        mesh=mesh,
        scratch_types=[
            pltpu.VMEM((nst, sst, _BT), jnp.int32),
            pltpu.VMEM(table_flat.shape, jnp.float32),
            pltpu.VMEM((_OB, depth, _PITCH), jnp.float32),
            pltpu.SemaphoreType.DMA((_OB,)),
        ],
        compiler_params=pltpu.CompilerParams(use_tc_tiling_on_sc=False,
                                             needs_layout_passes=False),
    )
    def run(idx_hbm, table_hbm, out_hbm, idx_vt, table_v, out_b, osem):
        wid = lax.axis_index("s") * _NC + lax.axis_index("c")

        pltpu.sync_copy(table_hbm, table_v)
        pltpu.sync_copy(idx_hbm.at[:, wid], idx_vt)

        def out_pairs(s, b):
            return [
                (out_b.at[b, pl.ds(dt * 8, 8), pl.ds(0, _BT)],
                 out_hbm.at[s, dt, wid])
                for dt in range(ndt)
            ]

        def fire_out(s, b):
            for src, dst in out_pairs(s, b):
                pltpu.async_copy(src, dst, osem.at[b])

        def wait_out(s, b):
            for src, dst in out_pairs(s, b):
                pltpu.make_async_copy(src, dst, osem.at[b]).wait()

        iota = lax.iota(jnp.int32, _L)

        @pl.loop(0, seq)
        def _step(s):
            ob = lax.rem(s, _OB)
            st = lax.div(s, sst)
            ss = lax.rem(s, sst)

            @pl.when(s >= _OB)
            def _drain():
                wait_out(s - _OB, ob)

            dst = out_b.at[ob]
            idxv = [idx_vt[st, ss, pl.ds(g * _L, _L)]
                    for g in range(_BT // _L)]
            for b in range(_BT):
                bc = _bcast_lane(idxv[b // _L], b % _L)
                addr = lax.shift_left(bc, _SH) | iota
                row = plsc.load_gather(table_v, [addr])
                plsc.store_scatter(dst, [iota, jnp.full((_L,), b, jnp.int32)],
                                   row)

            fire_out(s, ob)

        @pl.loop(seq - _OB, seq)
        def _final(s):
            wait_out(s, lax.rem(s, _OB))

    return run(idx4, table_flat)


def kernel(indices_tensor, table):
    batch, seq = indices_tensor.shape
    depth = table.shape[1]
    idx4 = (indices_tensor.astype(jnp.int32).T
            .reshape(seq // 8, 8, batch // _BT, _BT)
            .transpose(0, 2, 1, 3))
    out5 = _embed_gather(idx4, table.reshape(-1), depth)
    return out5.transpose(2, 4, 0, 1, 3).reshape(batch, seq, depth)


# speedup vs baseline: 2.0600x; 2.0600x over previous
import functools

import jax
import jax.numpy as jnp
from jax import lax
from jax.experimental import pallas as pl
from jax.experimental.pallas import tpu as pltpu
from jax.experimental.pallas import tpu_sc as plsc

_NC, _NS = 2, 16
_NW = _NC * _NS
_BT = 128
_OB = 4
_L = 16


def _embed_gather(idx4, table_t):
    nst, nbt, sst, _ = idx4.shape
    seq = nst * sst
    depth, vocab = table_t.shape
    ndt = depth // 8

    mesh = plsc.VectorSubcoreMesh(core_axis_name="c", subcore_axis_name="s")

    @functools.partial(
        pl.kernel,
        out_type=jax.ShapeDtypeStruct((seq, ndt, _NW, 8, _BT), jnp.float32),
        mesh=mesh,
        scratch_types=[
            pltpu.VMEM((nst, sst, _BT), jnp.int32),
            pltpu.VMEM((depth, vocab), jnp.float32),
            pltpu.VMEM((_OB, ndt, 8, _BT), jnp.float32),
            pltpu.SemaphoreType.DMA((_OB,)),
        ],
        compiler_params=pltpu.CompilerParams(use_tc_tiling_on_sc=False,
                                             needs_layout_passes=False),
    )
    def run(idx_hbm, table_hbm, out_hbm, idx_vt, table_v, out_b, osem):
        wid = lax.axis_index("s") * _NC + lax.axis_index("c")

        pltpu.sync_copy(table_hbm, table_v)
        pltpu.sync_copy(idx_hbm.at[:, wid], idx_vt)

        def fire_out(s, b):
            pltpu.async_copy(out_b.at[b], out_hbm.at[s, :, wid], osem.at[b])

        def wait_out(s, b):
            pltpu.make_async_copy(out_b.at[b], out_hbm.at[s, :, wid],
                                  osem.at[b]).wait()

        @pl.loop(0, seq, unroll=2)
        def _step(s):
            ob = lax.rem(s, _OB)

            @pl.when(s >= _OB)
            def _drain():
                wait_out(s - _OB, ob)

            st = lax.div(s, sst)
            ss = lax.rem(s, sst)
            idxv = [idx_vt[st, ss, pl.ds(g * _L, _L)]
                    for g in range(_BT // _L)]
            for d in range(depth):
                cols = jnp.full((_L,), d, jnp.int32)
                vals = [plsc.load_gather(table_v, [cols, idxv[g]])
                        for g in range(_BT // _L)]
                for g in range(_BT // _L):
                    out_b[ob, d // 8, d % 8, pl.ds(g * _L, _L)] = vals[g]

            fire_out(s, ob)

        @pl.loop(seq - _OB, seq)
        def _final(s):
            wait_out(s, lax.rem(s, _OB))

    return run(idx4, table_t)


def kernel(indices_tensor, table):
    batch, seq = indices_tensor.shape
    depth = table.shape[1]
    idx4 = (indices_tensor.astype(jnp.int32).T
            .reshape(seq // 8, 8, batch // _BT, _BT)
            .transpose(0, 2, 1, 3))
    out5 = _embed_gather(idx4, table.T)
    return out5.transpose(2, 4, 0, 1, 3).reshape(batch, seq, depth)


# speedup vs baseline: 2.0849x; 1.0121x over previous
import functools

import jax
import jax.numpy as jnp
from jax import lax
from jax.experimental import pallas as pl
from jax.experimental.pallas import tpu as pltpu
from jax.experimental.pallas import tpu_sc as plsc

_NC, _NS = 2, 16
_NW = _NC * _NS
_BT = 128
_OB = 4
_L = 16


def _embed_gather(idx4, table_t):
    nst, nbt, sst, _ = idx4.shape
    seq = nst * sst
    depth, vocab = table_t.shape
    ndt = depth // 8

    mesh = plsc.VectorSubcoreMesh(core_axis_name="c", subcore_axis_name="s")

    @functools.partial(
        pl.kernel,
        out_type=jax.ShapeDtypeStruct((seq, ndt, _NW, 8, _BT), jnp.float32),
        mesh=mesh,
        scratch_types=[
            pltpu.VMEM((nst, sst, _BT), jnp.int32),
            pltpu.VMEM((depth, vocab), jnp.float32),
            pltpu.VMEM((_OB, ndt, 8, _BT), jnp.float32),
            pltpu.SemaphoreType.DMA((_OB,)),
        ],
        compiler_params=pltpu.CompilerParams(use_tc_tiling_on_sc=False,
                                             needs_layout_passes=False),
    )
    def run(idx_hbm, table_hbm, out_hbm, idx_vt, table_v, out_b, osem):
        wid = lax.axis_index("s") * _NC + lax.axis_index("c")

        pltpu.sync_copy(table_hbm, table_v)
        pltpu.sync_copy(idx_hbm.at[:, wid], idx_vt)

        def fire_out(s, b):
            pltpu.async_copy(out_b.at[b], out_hbm.at[s, :, wid], osem.at[b])

        def wait_out(s, b):
            pltpu.make_async_copy(out_b.at[b], out_hbm.at[s, :, wid],
                                  osem.at[b]).wait()

        @pl.loop(0, seq)
        def _step(s):
            ob = lax.rem(s, _OB)

            @pl.when(s >= _OB)
            def _drain():
                wait_out(s - _OB, ob)

            st = lax.div(s, sst)
            ss = lax.rem(s, sst)
            idxv = [idx_vt[st, ss, pl.ds(g * _L, _L)]
                    for g in range(_BT // _L)]
            for d in range(depth):
                cols = jnp.full((_L,), d, jnp.int32)
                vals = [plsc.load_gather(table_v, [cols, idxv[g]])
                        for g in range(_BT // _L)]
                for g in range(_BT // _L):
                    out_b[ob, d // 8, d % 8, pl.ds(g * _L, _L)] = vals[g]

            fire_out(s, ob)

        @pl.loop(seq - _OB, seq)
        def _final(s):
            wait_out(s, lax.rem(s, _OB))

    return run(idx4, table_t)


def kernel(indices_tensor, table):
    batch, seq = indices_tensor.shape
    depth = table.shape[1]
    idx4 = (indices_tensor.astype(jnp.int32).T
            .reshape(seq // 8, 8, batch // _BT, _BT)
            .transpose(0, 2, 1, 3))
    out5 = _embed_gather(idx4, table.T)
    return out5.transpose(2, 4, 0, 1, 3).reshape(batch, seq, depth)
